# baseline (speedup 1.0000x reference)
import functools

import jax
import jax.numpy as jnp
from jax.experimental import pallas as pl
from jax.experimental.pallas import tpu as pltpu

_H, _W = 496, 432
_PT = 160


def _pfn_kernel(ft_ref, w_ref, xmax_ref, xmin_ref, s_ref, s2_ref):
    j = pl.program_id(1)
    ftb = ft_ref[0]
    a2 = ftb.reshape(9, _PT * 100)
    x = jnp.dot(w_ref[:, :], a2, preferred_element_type=jnp.float32)
    x3 = x.reshape(64, _PT, 100)
    xmax_ref[0, 0] = jnp.max(x3, axis=2)
    xmin_ref[0, 0] = jnp.min(x3, axis=2)
    ps = jnp.sum(x, axis=1).reshape(1, 64)
    ps2 = jnp.sum(x * x, axis=1).reshape(1, 64)

    @pl.when(j == 0)
    def _init():
        s_ref[0] = ps
        s2_ref[0] = ps2

    @pl.when(j != 0)
    def _acc():
        s_ref[0] += ps
        s2_ref[0] += ps2


def _scatter_kernel(ys_ref, xs_ref, xmax_ref, xmin_ref, a_ref, b_ref,
                    io_ref, out_ref):
    del ys_ref, xs_ref, io_ref
    a = a_ref[0, 0, :]
    b = b_ref[0, 0, :]
    sel = jnp.where(a >= 0.0, xmax_ref[0, 0, :], xmin_ref[0, 0, :])
    out_ref[0, 0, 0, :] = jnp.maximum(a * sel + b, 0.0)


@jax.jit
def kernel(feats, coords, W, gamma, beta):
    B, P, N, K = feats.shape
    C = W.shape[0]
    nj = P // _PT

    ft = feats.transpose(0, 3, 1, 2)
    xmax_t, xmin_t, s, s2 = pl.pallas_call(
        _pfn_kernel,
        grid=(B, nj),
        in_specs=[
            pl.BlockSpec((1, K, _PT, N), lambda b, j: (b, 0, j, 0)),
            pl.BlockSpec((C, K), lambda b, j: (0, 0)),
        ],
        out_specs=[
            pl.BlockSpec((1, 1, C, _PT), lambda b, j: (b, j, 0, 0)),
            pl.BlockSpec((1, 1, C, _PT), lambda b, j: (b, j, 0, 0)),
            pl.BlockSpec((1, 1, C), lambda b, j: (b, 0, 0)),
            pl.BlockSpec((1, 1, C), lambda b, j: (b, 0, 0)),
        ],
        out_shape=[
            jax.ShapeDtypeStruct((B, nj, C, _PT), jnp.float32),
            jax.ShapeDtypeStruct((B, nj, C, _PT), jnp.float32),
            jax.ShapeDtypeStruct((B, 1, C), jnp.float32),
            jax.ShapeDtypeStruct((B, 1, C), jnp.float32),
        ],
        compiler_params=pltpu.CompilerParams(
            dimension_semantics=("arbitrary", "arbitrary")),
    )(ft, W)

    cnt = jnp.float32(P * N)
    mean = s[:, 0, :] / cnt
    var = s2[:, 0, :] / cnt - mean * mean
    a_aff = gamma / jnp.sqrt(var + 1e-3)
    b_aff = beta - mean * a_aff

    xmax_r = xmax_t.transpose(0, 1, 3, 2).reshape(B * P, 1, C)
    xmin_r = xmin_t.transpose(0, 1, 3, 2).reshape(B * P, 1, C)
    a_r = a_aff.reshape(B, 1, C)
    b_r = b_aff.reshape(B, 1, C)
    ys = coords[..., 0].reshape(B * P).astype(jnp.int32)
    xs = coords[..., 1].reshape(B * P).astype(jnp.int32)
    canvas0 = jnp.zeros((B, _H * _W, 1, C), jnp.float32)

    canvas = pl.pallas_call(
        _scatter_kernel,
        grid_spec=pltpu.PrefetchScalarGridSpec(
            num_scalar_prefetch=2,
            grid=(B * P,),
            in_specs=[
                pl.BlockSpec((1, 1, C), lambda i, ys, xs: (i, 0, 0)),
                pl.BlockSpec((1, 1, C), lambda i, ys, xs: (i, 0, 0)),
                pl.BlockSpec((1, 1, C), lambda i, ys, xs: (i // P, 0, 0)),
                pl.BlockSpec((1, 1, C), lambda i, ys, xs: (i // P, 0, 0)),
                pl.BlockSpec((1, 1, 1, C),
                             lambda i, ys, xs: (i // P, ys[i] * _W + xs[i], 0, 0)),
            ],
            out_specs=pl.BlockSpec((1, 1, 1, C),
                                   lambda i, ys, xs: (i // P, ys[i] * _W + xs[i], 0, 0)),
        ),
        out_shape=jax.ShapeDtypeStruct((B, _H * _W, 1, C), jnp.float32),
        input_output_aliases={6: 0},
        compiler_params=pltpu.CompilerParams(
            dimension_semantics=("arbitrary",)),
    )(ys, xs, xmax_r, xmin_r, a_r, b_r, canvas0)

    return canvas.reshape(B, _H, _W, C).transpose(0, 3, 1, 2)


# speedup vs baseline: 1.0330x; 1.0330x over previous
import functools

import jax
import jax.numpy as jnp
from jax.experimental import pallas as pl
from jax.experimental.pallas import tpu as pltpu

_H, _W = 496, 432
_PT = 160


def _pfn_kernel(ft_ref, w_ref, xmax_ref, xmin_ref, s_ref, s2_ref):
    j = pl.program_id(1)
    ftb = ft_ref[0]
    a2 = ftb.reshape(9, _PT * 100)
    x = jnp.dot(w_ref[:, :], a2, preferred_element_type=jnp.float32)
    x3 = x.reshape(64, _PT, 100)
    xmax_ref[0, 0] = jnp.max(x3, axis=2)
    xmin_ref[0, 0] = jnp.min(x3, axis=2)
    ps = jnp.sum(x, axis=1).reshape(1, 64)
    ps2 = jnp.sum(x * x, axis=1).reshape(1, 64)

    @pl.when(j == 0)
    def _init():
        s_ref[0] = ps
        s2_ref[0] = ps2

    @pl.when(j != 0)
    def _acc():
        s_ref[0] += ps
        s2_ref[0] += ps2


def _affine_kernel(xmax_ref, xmin_ref, a_ref, b_ref, pf_ref):
    a = a_ref[0, :, :]
    b = b_ref[0, :, :]
    xm = xmax_ref[:, 0, :]
    xn = xmin_ref[:, 0, :]
    sel = jnp.where(a >= 0.0, xm, xn)
    pf_ref[:, 0, :] = jnp.maximum(sel * a + b, 0.0)


def _scatter_kernel(ys_ref, xs_ref, pf_ref, io_ref, out_ref):
    del ys_ref, xs_ref, io_ref
    out_ref[0, 0, 0, :] = pf_ref[0, 0, :]


@jax.jit
def kernel(feats, coords, W, gamma, beta):
    B, P, N, K = feats.shape
    C = W.shape[0]
    nj = P // _PT

    ft = feats.transpose(0, 3, 1, 2)
    xmax_t, xmin_t, s, s2 = pl.pallas_call(
        _pfn_kernel,
        grid=(B, nj),
        in_specs=[
            pl.BlockSpec((1, K, _PT, N), lambda b, j: (b, 0, j, 0)),
            pl.BlockSpec((C, K), lambda b, j: (0, 0)),
        ],
        out_specs=[
            pl.BlockSpec((1, 1, C, _PT), lambda b, j: (b, j, 0, 0)),
            pl.BlockSpec((1, 1, C, _PT), lambda b, j: (b, j, 0, 0)),
            pl.BlockSpec((1, 1, C), lambda b, j: (b, 0, 0)),
            pl.BlockSpec((1, 1, C), lambda b, j: (b, 0, 0)),
        ],
        out_shape=[
            jax.ShapeDtypeStruct((B, nj, C, _PT), jnp.float32),
            jax.ShapeDtypeStruct((B, nj, C, _PT), jnp.float32),
            jax.ShapeDtypeStruct((B, 1, C), jnp.float32),
            jax.ShapeDtypeStruct((B, 1, C), jnp.float32),
        ],
        compiler_params=pltpu.CompilerParams(
            dimension_semantics=("arbitrary", "arbitrary")),
    )(ft, W)

    cnt = jnp.float32(P * N)
    mean = s[:, 0, :] / cnt
    var = s2[:, 0, :] / cnt - mean * mean
    a_aff = gamma / jnp.sqrt(var + 1e-3)
    b_aff = beta - mean * a_aff

    a_r = a_aff.reshape(B, 1, C)
    b_r = b_aff.reshape(B, 1, C)
    xmax_r = xmax_t.transpose(0, 1, 3, 2).reshape(B * P, 1, C)
    xmin_r = xmin_t.transpose(0, 1, 3, 2).reshape(B * P, 1, C)
    PR = 800
    npr = P // PR
    pf_r = pl.pallas_call(
        _affine_kernel,
        grid=(B * P // PR,),
        in_specs=[
            pl.BlockSpec((PR, 1, C), lambda i: (i, 0, 0)),
            pl.BlockSpec((PR, 1, C), lambda i: (i, 0, 0)),
            pl.BlockSpec((1, 1, C), lambda i: (i // npr, 0, 0)),
            pl.BlockSpec((1, 1, C), lambda i: (i // npr, 0, 0)),
        ],
        out_specs=pl.BlockSpec((PR, 1, C), lambda i: (i, 0, 0)),
        out_shape=jax.ShapeDtypeStruct((B * P, 1, C), jnp.float32),
    )(xmax_r, xmin_r, a_r, b_r)

    ys = coords[..., 0].reshape(B * P).astype(jnp.int32)
    xs = coords[..., 1].reshape(B * P).astype(jnp.int32)
    canvas0 = jnp.zeros((B, _H * _W, 1, C), jnp.float32)

    canvas = pl.pallas_call(
        _scatter_kernel,
        grid_spec=pltpu.PrefetchScalarGridSpec(
            num_scalar_prefetch=2,
            grid=(B * P,),
            in_specs=[
                pl.BlockSpec((1, 1, C), lambda i, ys, xs: (i, 0, 0)),
                pl.BlockSpec(memory_space=pl.ANY),
            ],
            out_specs=pl.BlockSpec((1, 1, 1, C),
                                   lambda i, ys, xs: (i // P, ys[i] * _W + xs[i], 0, 0)),
        ),
        out_shape=jax.ShapeDtypeStruct((B, _H * _W, 1, C), jnp.float32),
        input_output_aliases={3: 0},
        compiler_params=pltpu.CompilerParams(
            dimension_semantics=("arbitrary",)),
    )(ys, xs, pf_r, canvas0)

    return canvas.reshape(B, _H, _W, C).transpose(0, 3, 1, 2)
